# Initial kernel scaffold; baseline (speedup 1.0000x reference)
import jax
import jax.numpy as jnp
from jax.experimental import pallas as pl


def kernel(x, ids, edge_index, eW0, eb0, eW1, eb1, eW2, eb2, eW3, eb3, nW0, nb0, nW1, nb1, nW2, nb2, nW3, nb3):
    raise NotImplementedError("write your pallas kernel here")


# speedup vs baseline: 2.4318x; 2.4318x over previous
import functools

import jax
import jax.numpy as jnp
from jax import lax
from jax.experimental import pallas as pl
from jax.experimental.pallas import tpu as pltpu
from jax.experimental.pallas import tpu_sc as plsc

N = 10000
DZ = -0.0106666666666666
DT = 0.01

NC = 2
NS = 16
NW = NC * NS
CH = 128
NP = N + 112
ROWS_PER_SUB = NP // NS


def _prep_kernel(x_ref, ids_ref, o_ref):
    x = x_ref[...]
    ids = ids_ref[...]
    xb = jnp.concatenate(
        [x[:, 0:2], x[:, 2:3] - ids * jnp.float32(DZ), x[:, 3:6], ids,
         jnp.zeros((x.shape[0], 9), jnp.float32)], axis=1)
    o_ref[...] = jnp.concatenate(
        [xb, jnp.zeros((16, 16), jnp.float32)], axis=0)


def _gather_kernel(K, xp_hbm, send_hbm, recv_hbm, out_hbm,
                   sidx, ridx, srows, rrows, sem0, sem1):
    wid = lax.axis_index("s") * NC + lax.axis_index("c")

    def chunk(k, _):
        off = wid * (CH * K) + k * CH
        pltpu.sync_copy(send_hbm.at[pl.ds(off, CH)], sidx)
        pltpu.sync_copy(recv_hbm.at[pl.ds(off, CH)], ridx)
        cp0 = pltpu.async_copy(xp_hbm.at[sidx], srows, sem0)
        cp1 = pltpu.async_copy(xp_hbm.at[ridx], rrows, sem1)
        cp0.wait()
        cp1.wait()
        pltpu.sync_copy(rrows, out_hbm.at[0, pl.ds(off, CH)])
        pltpu.sync_copy(srows, out_hbm.at[1, pl.ds(off, CH)])
        return 0

    lax.fori_loop(0, K, chunk, 0)


def _edge_mlp_kernel(r_ref, s_ref, w0, b0, w1, b1, w2, b2, w3, b3, o_ref):
    d = r_ref[...] - s_ref[...]
    h = jnp.dot(d, w0[...], preferred_element_type=jnp.float32)
    h = jnp.maximum(h + b0[...], 0.0)
    h = jnp.dot(h, w1[...], preferred_element_type=jnp.float32)
    h = jnp.maximum(h + b1[...], 0.0)
    h = jnp.dot(h, w2[...], preferred_element_type=jnp.float32)
    h = jnp.maximum(h + b2[...], 0.0)
    h = jnp.dot(h, w3[...], preferred_element_type=jnp.float32)
    o_ref[...] = h + b3[...]


def _scatter_kernel(K, marsh_hbm, recv_hbm, zeros_hbm, out_hbm,
                    shared, mbuf, idxv):
    c = lax.axis_index("c")
    s = lax.axis_index("s")
    wid = s * NC + c
    r0 = s * ROWS_PER_SUB
    pltpu.sync_copy(zeros_hbm, shared.at[pl.ds(r0, ROWS_PER_SUB)])
    plsc.subcore_barrier()

    def chunk(k, _):
        off = wid * (CH * K) + k * CH
        pltpu.sync_copy(recv_hbm.at[pl.ds(off, CH)], idxv)
        pltpu.sync_copy(marsh_hbm.at[pl.ds(off, CH)], mbuf)
        pltpu.sync_copy(mbuf, shared.at[idxv], add=True)
        return 0

    lax.fori_loop(0, K, chunk, 0)
    plsc.subcore_barrier()
    pltpu.sync_copy(shared.at[pl.ds(r0, ROWS_PER_SUB)],
                    out_hbm.at[c, pl.ds(r0, ROWS_PER_SUB)])


def _node_mlp_kernel(x_ref, ids_ref, p0_ref, p1_ref,
                     w0a, w0b, b0, w1, b1, w2, b2, w3, b3, o_ref):
    x = x_ref[...]
    ids = ids_ref[...]
    rest = jnp.concatenate(
        [jnp.zeros_like(ids), jnp.zeros_like(ids), ids * jnp.float32(DZ),
         jnp.zeros_like(ids), jnp.zeros_like(ids), jnp.zeros_like(ids)],
        axis=1)
    xb = jnp.concatenate([x - rest, ids, jnp.zeros_like(ids)], axis=1)
    aggr = p0_ref[...] + p1_ref[...]
    h = (jnp.dot(xb, w0a[...], preferred_element_type=jnp.float32)
         + jnp.dot(aggr, w0b[...], preferred_element_type=jnp.float32))
    h = jnp.maximum(h + b0[...], 0.0)
    h = jnp.dot(h, w1[...], preferred_element_type=jnp.float32)
    h = jnp.maximum(h + b1[...], 0.0)
    h = jnp.dot(h, w2[...], preferred_element_type=jnp.float32)
    h = jnp.maximum(h + b2[...], 0.0)
    d8 = jnp.dot(h, w3[...], preferred_element_type=jnp.float32) + b3[...]
    v = x[:, 3:6] + d8[:, 0:3]
    xn = x[:, 0:3] + v * jnp.float32(DT)
    o_ref[...] = jnp.concatenate([xn, v, jnp.zeros_like(v[:, 0:2])], axis=1)


def kernel(x, ids, edge_index, eW0, eb0, eW1, eb1, eW2, eb2, eW3, eb3,
           nW0, nb0, nW1, nb1, nW2, nb2, nW3, nb3):
    E = edge_index.shape[1]
    K = -(-E // (NW * CH))
    E_pad = NW * CH * K

    send = edge_index[0]
    recv = edge_index[1]
    pad = jnp.full((E_pad - E,), N, jnp.int32)
    send_p = jnp.concatenate([send, pad])
    recv_p = jnp.concatenate([recv, pad])

    xp = pl.pallas_call(
        _prep_kernel,
        out_shape=jax.ShapeDtypeStruct((N + 16, 16), jnp.float32),
    )(x, ids)

    mesh = plsc.VectorSubcoreMesh(core_axis_name="c", subcore_axis_name="s",
                                  num_cores=NC, num_subcores=NS)
    gather = functools.partial(
        pl.kernel,
        out_type=jax.ShapeDtypeStruct((2, E_pad, 16), jnp.float32),
        mesh=mesh,
        compiler_params=pltpu.CompilerParams(use_tc_tiling_on_sc=False),
        scratch_types=[
            pltpu.VMEM((CH,), jnp.int32),
            pltpu.VMEM((CH,), jnp.int32),
            pltpu.VMEM((CH, 16), jnp.float32),
            pltpu.VMEM((CH, 16), jnp.float32),
            pltpu.SemaphoreType.DMA,
            pltpu.SemaphoreType.DMA,
        ],
    )(functools.partial(_gather_kernel, K))
    planes = gather(xp, send_p, recv_p)
    rplane = planes[0]
    splane = planes[1]

    eW0eff = jnp.pad(eW0, ((0, 9), (0, 0)))

    BE = 2048
    grid = E_pad // BE
    marsh = pl.pallas_call(
        _edge_mlp_kernel,
        grid=(grid,),
        in_specs=[
            pl.BlockSpec((BE, 16), lambda i: (i, 0)),
            pl.BlockSpec((BE, 16), lambda i: (i, 0)),
            pl.BlockSpec((16, 128), lambda i: (0, 0)),
            pl.BlockSpec((1, 128), lambda i: (0, 0)),
            pl.BlockSpec((128, 128), lambda i: (0, 0)),
            pl.BlockSpec((1, 128), lambda i: (0, 0)),
            pl.BlockSpec((128, 128), lambda i: (0, 0)),
            pl.BlockSpec((1, 128), lambda i: (0, 0)),
            pl.BlockSpec((128, 128), lambda i: (0, 0)),
            pl.BlockSpec((1, 128), lambda i: (0, 0)),
        ],
        out_specs=pl.BlockSpec((BE, 128), lambda i: (i, 0)),
        out_shape=jax.ShapeDtypeStruct((E_pad, 128), jnp.float32),
    )(rplane, splane, eW0eff, eb0.reshape(1, 128), eW1, eb1.reshape(1, 128),
      eW2, eb2.reshape(1, 128), eW3, eb3.reshape(1, 128))

    zeros_rows = jnp.zeros((ROWS_PER_SUB, 128), jnp.float32)
    scatter = functools.partial(
        pl.kernel,
        out_type=jax.ShapeDtypeStruct((NC, NP, 128), jnp.float32),
        mesh=mesh,
        scratch_types=[
            pltpu.VMEM_SHARED((NP, 128), jnp.float32),
            pltpu.VMEM((CH, 128), jnp.float32),
            pltpu.VMEM((CH,), jnp.int32),
        ],
    )(functools.partial(_scatter_kernel, K))
    partials = scatter(marsh, recv_p, zeros_rows)

    BN = 2000
    p0 = partials[0, :N]
    p1 = partials[1, :N]
    nW0a = jnp.pad(nW0[:7], ((0, 1), (0, 0)))
    nW0b = nW0[7:]
    nW3p = jnp.pad(nW3, ((0, 0), (0, 5)))
    nb3p = jnp.pad(nb3, ((0, 5),)).reshape(1, 8)
    out8 = pl.pallas_call(
        _node_mlp_kernel,
        grid=(N // BN,),
        in_specs=[
            pl.BlockSpec((BN, 6), lambda i: (i, 0)),
            pl.BlockSpec((BN, 1), lambda i: (i, 0)),
            pl.BlockSpec((BN, 128), lambda i: (i, 0)),
            pl.BlockSpec((BN, 128), lambda i: (i, 0)),
            pl.BlockSpec((8, 128), lambda i: (0, 0)),
            pl.BlockSpec((128, 128), lambda i: (0, 0)),
            pl.BlockSpec((1, 128), lambda i: (0, 0)),
            pl.BlockSpec((128, 128), lambda i: (0, 0)),
            pl.BlockSpec((1, 128), lambda i: (0, 0)),
            pl.BlockSpec((128, 128), lambda i: (0, 0)),
            pl.BlockSpec((1, 128), lambda i: (0, 0)),
            pl.BlockSpec((128, 8), lambda i: (0, 0)),
            pl.BlockSpec((1, 8), lambda i: (0, 0)),
        ],
        out_specs=pl.BlockSpec((BN, 8), lambda i: (i, 0)),
        out_shape=jax.ShapeDtypeStruct((N, 8), jnp.float32),
    )(x, ids, p0, p1, nW0a, nW0b, nb0.reshape(1, 128),
      nW1, nb1.reshape(1, 128), nW2, nb2.reshape(1, 128), nW3p, nb3p)
    return out8[:, :6]


# speedup vs baseline: 2.5247x; 1.0382x over previous
import functools

import jax
import jax.numpy as jnp
from jax import lax
from jax.experimental import pallas as pl
from jax.experimental.pallas import tpu as pltpu
from jax.experimental.pallas import tpu_sc as plsc

N = 10000
DZ = -0.0106666666666666
DT = 0.01

NC = 2
NS = 16
NW = NC * NS
CH = 128
NP = N + 112
ROWS_PER_SUB = NP // NS


def _prep_kernel(x_ref, ids_ref, o_ref):
    x = x_ref[...]
    ids = ids_ref[...]
    xb = jnp.concatenate(
        [x[:, 0:2], x[:, 2:3] - ids * jnp.float32(DZ), x[:, 3:6], ids,
         jnp.zeros((x.shape[0], 9), jnp.float32)], axis=1)
    o_ref[...] = jnp.concatenate(
        [xb, jnp.zeros((16, 16), jnp.float32)], axis=0)


def _gather_kernel(K, xp_hbm, send_hbm, recv_hbm, out_hbm,
                   sidx, ridx, srows, rrows, sem0, sem1):
    wid = lax.axis_index("s") * NC + lax.axis_index("c")

    def chunk(k, _):
        off = wid * (CH * K) + k * CH
        pltpu.sync_copy(send_hbm.at[pl.ds(off, CH)], sidx)
        pltpu.sync_copy(recv_hbm.at[pl.ds(off, CH)], ridx)
        cp0 = pltpu.async_copy(xp_hbm.at[sidx], srows, sem0)
        cp1 = pltpu.async_copy(xp_hbm.at[ridx], rrows, sem1)
        cp0.wait()
        cp1.wait()
        pltpu.sync_copy(rrows, out_hbm.at[0, pl.ds(off, CH)])
        pltpu.sync_copy(srows, out_hbm.at[1, pl.ds(off, CH)])
        return 0

    lax.fori_loop(0, K, chunk, 0)


def _edge_mlp_kernel(r_ref, s_ref, w0, b0, w1, b1, w2, b2, w3, b3, o_ref):
    d = r_ref[...] - s_ref[...]
    h = jnp.concatenate(
        [jnp.dot(d[:, 16 * q:16 * q + 16], w0[...],
                 preferred_element_type=jnp.float32) for q in range(8)],
        axis=0)
    h = jnp.maximum(h + b0[...], 0.0)
    h = jnp.dot(h, w1[...], preferred_element_type=jnp.float32)
    h = jnp.maximum(h + b1[...], 0.0)
    h = jnp.dot(h, w2[...], preferred_element_type=jnp.float32)
    h = jnp.maximum(h + b2[...], 0.0)
    h = jnp.dot(h, w3[...], preferred_element_type=jnp.float32)
    o_ref[...] = h + b3[...]


def _scatter_kernel(K, marsh_hbm, recv_hbm, zeros_hbm, out_hbm,
                    shared, mbuf, idxv):
    c = lax.axis_index("c")
    s = lax.axis_index("s")
    wid = s * NC + c
    r0 = s * ROWS_PER_SUB
    pltpu.sync_copy(zeros_hbm, shared.at[pl.ds(r0, ROWS_PER_SUB)])
    plsc.subcore_barrier()

    def chunk(k, _):
        off = wid * (CH * K) + k * CH
        pltpu.sync_copy(recv_hbm.at[pl.ds(off, CH)], idxv)
        pltpu.sync_copy(marsh_hbm.at[pl.ds(off, CH)], mbuf)
        pltpu.sync_copy(mbuf, shared.at[idxv], add=True)
        return 0

    lax.fori_loop(0, K, chunk, 0)
    plsc.subcore_barrier()
    pltpu.sync_copy(shared.at[pl.ds(r0, ROWS_PER_SUB)],
                    out_hbm.at[c, pl.ds(r0, ROWS_PER_SUB)])


def _node_mlp_kernel(x_ref, ids_ref, p0_ref, p1_ref,
                     w0a, w0b, b0, w1, b1, w2, b2, w3, b3, o_ref):
    x = x_ref[...]
    ids = ids_ref[...]
    rest = jnp.concatenate(
        [jnp.zeros_like(ids), jnp.zeros_like(ids), ids * jnp.float32(DZ),
         jnp.zeros_like(ids), jnp.zeros_like(ids), jnp.zeros_like(ids)],
        axis=1)
    xb = jnp.concatenate([x - rest, ids, jnp.zeros_like(ids)], axis=1)
    aggr = p0_ref[...] + p1_ref[...]
    h = (jnp.dot(xb, w0a[...], preferred_element_type=jnp.float32)
         + jnp.dot(aggr, w0b[...], preferred_element_type=jnp.float32))
    h = jnp.maximum(h + b0[...], 0.0)
    h = jnp.dot(h, w1[...], preferred_element_type=jnp.float32)
    h = jnp.maximum(h + b1[...], 0.0)
    h = jnp.dot(h, w2[...], preferred_element_type=jnp.float32)
    h = jnp.maximum(h + b2[...], 0.0)
    d8 = jnp.dot(h, w3[...], preferred_element_type=jnp.float32) + b3[...]
    v = x[:, 3:6] + d8[:, 0:3]
    xn = x[:, 0:3] + v * jnp.float32(DT)
    o_ref[...] = jnp.concatenate([xn, v, jnp.zeros_like(v[:, 0:2])], axis=1)


def kernel(x, ids, edge_index, eW0, eb0, eW1, eb1, eW2, eb2, eW3, eb3,
           nW0, nb0, nW1, nb1, nW2, nb2, nW3, nb3):
    E = edge_index.shape[1]
    K = -(-E // (NW * CH))
    E_pad = NW * CH * K

    send = edge_index[0]
    recv = edge_index[1]
    pad = jnp.full((E_pad - E,), N, jnp.int32)
    send_p = jnp.concatenate([send, pad])
    recv_p = jnp.concatenate([recv, pad])

    xp = pl.pallas_call(
        _prep_kernel,
        out_shape=jax.ShapeDtypeStruct((N + 16, 16), jnp.float32),
    )(x, ids)

    mesh = plsc.VectorSubcoreMesh(core_axis_name="c", subcore_axis_name="s",
                                  num_cores=NC, num_subcores=NS)
    gather = functools.partial(
        pl.kernel,
        out_type=jax.ShapeDtypeStruct((2, E_pad, 16), jnp.float32),
        mesh=mesh,
        compiler_params=pltpu.CompilerParams(use_tc_tiling_on_sc=False),
        scratch_types=[
            pltpu.VMEM((CH,), jnp.int32),
            pltpu.VMEM((CH,), jnp.int32),
            pltpu.VMEM((CH, 16), jnp.float32),
            pltpu.VMEM((CH, 16), jnp.float32),
            pltpu.SemaphoreType.DMA,
            pltpu.SemaphoreType.DMA,
        ],
    )(functools.partial(_gather_kernel, K))
    planes = gather(xp, send_p, recv_p)
    pk = planes.reshape(2, E_pad // 8, 128)
    rplane = pk[0]
    splane = pk[1]

    eW0eff = jnp.pad(eW0, ((0, 9), (0, 0)))

    BE = 2048
    grid = E_pad // BE
    marsh = pl.pallas_call(
        _edge_mlp_kernel,
        grid=(grid,),
        in_specs=[
            pl.BlockSpec((BE // 8, 128), lambda i: (i, 0)),
            pl.BlockSpec((BE // 8, 128), lambda i: (i, 0)),
            pl.BlockSpec((16, 128), lambda i: (0, 0)),
            pl.BlockSpec((1, 128), lambda i: (0, 0)),
            pl.BlockSpec((128, 128), lambda i: (0, 0)),
            pl.BlockSpec((1, 128), lambda i: (0, 0)),
            pl.BlockSpec((128, 128), lambda i: (0, 0)),
            pl.BlockSpec((1, 128), lambda i: (0, 0)),
            pl.BlockSpec((128, 128), lambda i: (0, 0)),
            pl.BlockSpec((1, 128), lambda i: (0, 0)),
        ],
        out_specs=pl.BlockSpec((BE, 128), lambda i: (i, 0)),
        out_shape=jax.ShapeDtypeStruct((E_pad, 128), jnp.float32),
    )(rplane, splane, eW0eff, eb0.reshape(1, 128), eW1, eb1.reshape(1, 128),
      eW2, eb2.reshape(1, 128), eW3, eb3.reshape(1, 128))

    recv_perm = recv_p.reshape(E_pad // BE, BE // 8, 8)
    recv_perm = recv_perm.transpose(0, 2, 1).reshape(-1)
    zeros_rows = jnp.zeros((ROWS_PER_SUB, 128), jnp.float32)
    scatter = functools.partial(
        pl.kernel,
        out_type=jax.ShapeDtypeStruct((NC, NP, 128), jnp.float32),
        mesh=mesh,
        scratch_types=[
            pltpu.VMEM_SHARED((NP, 128), jnp.float32),
            pltpu.VMEM((CH, 128), jnp.float32),
            pltpu.VMEM((CH,), jnp.int32),
        ],
    )(functools.partial(_scatter_kernel, K))
    partials = scatter(marsh, recv_perm, zeros_rows)

    BN = 2000
    p0 = partials[0, :N]
    p1 = partials[1, :N]
    nW0a = jnp.pad(nW0[:7], ((0, 1), (0, 0)))
    nW0b = nW0[7:]
    nW3p = jnp.pad(nW3, ((0, 0), (0, 5)))
    nb3p = jnp.pad(nb3, ((0, 5),)).reshape(1, 8)
    out8 = pl.pallas_call(
        _node_mlp_kernel,
        grid=(N // BN,),
        in_specs=[
            pl.BlockSpec((BN, 6), lambda i: (i, 0)),
            pl.BlockSpec((BN, 1), lambda i: (i, 0)),
            pl.BlockSpec((BN, 128), lambda i: (i, 0)),
            pl.BlockSpec((BN, 128), lambda i: (i, 0)),
            pl.BlockSpec((8, 128), lambda i: (0, 0)),
            pl.BlockSpec((128, 128), lambda i: (0, 0)),
            pl.BlockSpec((1, 128), lambda i: (0, 0)),
            pl.BlockSpec((128, 128), lambda i: (0, 0)),
            pl.BlockSpec((1, 128), lambda i: (0, 0)),
            pl.BlockSpec((128, 128), lambda i: (0, 0)),
            pl.BlockSpec((1, 128), lambda i: (0, 0)),
            pl.BlockSpec((128, 8), lambda i: (0, 0)),
            pl.BlockSpec((1, 8), lambda i: (0, 0)),
        ],
        out_specs=pl.BlockSpec((BN, 8), lambda i: (i, 0)),
        out_shape=jax.ShapeDtypeStruct((N, 8), jnp.float32),
    )(x, ids, p0, p1, nW0a, nW0b, nb0.reshape(1, 128),
      nW1, nb1.reshape(1, 128), nW2, nb2.reshape(1, 128), nW3p, nb3p)
    return out8[:, :6]


# speedup vs baseline: 4.0159x; 1.5906x over previous
import functools

import jax
import jax.numpy as jnp
from jax import lax
from jax.experimental import pallas as pl
from jax.experimental.pallas import tpu as pltpu
from jax.experimental.pallas import tpu_sc as plsc

N = 10000
DZ = -0.0106666666666666
DT = 0.01

NC = 2
NS = 16
NW = NC * NS
CH = 128
NP = N + 112
ROWS_PER_SUB = NP // NS


def _prep_kernel(x_ref, ids_ref, o_ref):
    x = x_ref[...]
    ids = ids_ref[...]
    xb = jnp.concatenate(
        [x[:, 0:2], x[:, 2:3] - ids * jnp.float32(DZ), x[:, 3:6], ids,
         jnp.zeros((x.shape[0], 9), jnp.float32)], axis=1)
    o_ref[...] = jnp.concatenate(
        [xb, jnp.zeros((16, 16), jnp.float32)], axis=0)


def _gather_kernel(K, xp_hbm, send_hbm, recv_hbm, rout_hbm, sout_hbm,
                   sidx, ridx, srows, rrows, sem0, sem1):
    wid = lax.axis_index("s") * NC + lax.axis_index("c")

    def chunk(k, _):
        off = wid * (CH * K) + k * CH
        pltpu.sync_copy(send_hbm.at[pl.ds(off, CH)], sidx)
        pltpu.sync_copy(recv_hbm.at[pl.ds(off, CH)], ridx)
        cp0 = pltpu.async_copy(xp_hbm.at[sidx], srows, sem0)
        cp1 = pltpu.async_copy(xp_hbm.at[ridx], rrows, sem1)
        cp0.wait()
        cp1.wait()
        pltpu.sync_copy(rrows, rout_hbm.at[pl.ds(off, CH)])
        pltpu.sync_copy(srows, sout_hbm.at[pl.ds(off, CH)])
        return 0

    lax.fori_loop(0, K, chunk, 0)


def _edge_mlp_kernel(r_ref, s_ref, w0, b0, w1, b1, w2, b2, w3, b3, o_ref):
    d = r_ref[...] - s_ref[...]
    h = jnp.concatenate(
        [jnp.dot(d[:, 16 * q:16 * q + 16], w0[...],
                 preferred_element_type=jnp.float32) for q in range(8)],
        axis=0)
    h = jnp.maximum(h + b0[...], 0.0)
    h = jnp.dot(h, w1[...], preferred_element_type=jnp.float32)
    h = jnp.maximum(h + b1[...], 0.0)
    h = jnp.dot(h, w2[...], preferred_element_type=jnp.float32)
    h = jnp.maximum(h + b2[...], 0.0)
    h = jnp.dot(h, w3[...], preferred_element_type=jnp.float32)
    o_ref[...] = h + b3[...]


def _scatter_kernel(K, marsh_hbm, recv_hbm, zeros_hbm, out_hbm,
                    shared, mbuf, idxv):
    c = lax.axis_index("c")
    s = lax.axis_index("s")
    wid = s * NC + c
    r0 = s * ROWS_PER_SUB
    pltpu.sync_copy(zeros_hbm, shared.at[pl.ds(r0, ROWS_PER_SUB)])
    plsc.subcore_barrier()

    def chunk(k, _):
        off = wid * (CH * K) + k * CH
        pltpu.sync_copy(recv_hbm.at[pl.ds(off, CH)], idxv)
        pltpu.sync_copy(marsh_hbm.at[pl.ds(off, CH)], mbuf)
        pltpu.sync_copy(mbuf, shared.at[idxv], add=True)
        return 0

    lax.fori_loop(0, K, chunk, 0)
    plsc.subcore_barrier()
    pltpu.sync_copy(shared.at[pl.ds(r0, ROWS_PER_SUB)],
                    out_hbm.at[c, pl.ds(r0, ROWS_PER_SUB)])


def _node_mlp_kernel(x_ref, ids_ref, p0_ref, p1_ref,
                     w0a, w0b, b0, w1, b1, w2, b2, w3, b3, o_ref):
    x = x_ref[...]
    ids = ids_ref[...]
    rest = jnp.concatenate(
        [jnp.zeros_like(ids), jnp.zeros_like(ids), ids * jnp.float32(DZ),
         jnp.zeros_like(ids), jnp.zeros_like(ids), jnp.zeros_like(ids)],
        axis=1)
    xb = jnp.concatenate([x - rest, ids, jnp.zeros_like(ids)], axis=1)
    aggr = p0_ref[...] + p1_ref[...]
    h = (jnp.dot(xb, w0a[...], preferred_element_type=jnp.float32)
         + jnp.dot(aggr, w0b[...], preferred_element_type=jnp.float32))
    h = jnp.maximum(h + b0[...], 0.0)
    h = jnp.dot(h, w1[...], preferred_element_type=jnp.float32)
    h = jnp.maximum(h + b1[...], 0.0)
    h = jnp.dot(h, w2[...], preferred_element_type=jnp.float32)
    h = jnp.maximum(h + b2[...], 0.0)
    d8 = jnp.dot(h, w3[...], preferred_element_type=jnp.float32) + b3[...]
    v = x[:, 3:6] + d8[:, 0:3]
    xn = x[:, 0:3] + v * jnp.float32(DT)
    o_ref[...] = jnp.concatenate([xn, v, jnp.zeros_like(v[:, 0:2])], axis=1)


def kernel(x, ids, edge_index, eW0, eb0, eW1, eb1, eW2, eb2, eW3, eb3,
           nW0, nb0, nW1, nb1, nW2, nb2, nW3, nb3):
    E = edge_index.shape[1]
    K = -(-E // (NW * CH))
    E_pad = NW * CH * K

    send = edge_index[0]
    recv = edge_index[1]
    pad = jnp.full((E_pad - E,), N, jnp.int32)
    send_p = jnp.concatenate([send, pad])
    recv_p = jnp.concatenate([recv, pad])

    xp = pl.pallas_call(
        _prep_kernel,
        out_shape=jax.ShapeDtypeStruct((N + 16, 16), jnp.float32),
    )(x, ids)

    mesh = plsc.VectorSubcoreMesh(core_axis_name="c", subcore_axis_name="s",
                                  num_cores=NC, num_subcores=NS)
    gather = functools.partial(
        pl.kernel,
        out_type=[jax.ShapeDtypeStruct((E_pad, 16), jnp.float32),
                  jax.ShapeDtypeStruct((E_pad, 16), jnp.float32)],
        mesh=mesh,
        compiler_params=pltpu.CompilerParams(use_tc_tiling_on_sc=False),
        scratch_types=[
            pltpu.VMEM((CH,), jnp.int32),
            pltpu.VMEM((CH,), jnp.int32),
            pltpu.VMEM((CH, 16), jnp.float32),
            pltpu.VMEM((CH, 16), jnp.float32),
            pltpu.SemaphoreType.DMA,
            pltpu.SemaphoreType.DMA,
        ],
    )(functools.partial(_gather_kernel, K))
    rout, sout = gather(xp, send_p, recv_p)
    rplane = rout.reshape(E_pad // 8, 128)
    splane = sout.reshape(E_pad // 8, 128)

    eW0eff = jnp.pad(eW0, ((0, 9), (0, 0)))

    BE = 2048
    grid = E_pad // BE
    marsh = pl.pallas_call(
        _edge_mlp_kernel,
        grid=(grid,),
        in_specs=[
            pl.BlockSpec((BE // 8, 128), lambda i: (i, 0)),
            pl.BlockSpec((BE // 8, 128), lambda i: (i, 0)),
            pl.BlockSpec((16, 128), lambda i: (0, 0)),
            pl.BlockSpec((1, 128), lambda i: (0, 0)),
            pl.BlockSpec((128, 128), lambda i: (0, 0)),
            pl.BlockSpec((1, 128), lambda i: (0, 0)),
            pl.BlockSpec((128, 128), lambda i: (0, 0)),
            pl.BlockSpec((1, 128), lambda i: (0, 0)),
            pl.BlockSpec((128, 128), lambda i: (0, 0)),
            pl.BlockSpec((1, 128), lambda i: (0, 0)),
        ],
        out_specs=pl.BlockSpec((BE, 128), lambda i: (i, 0)),
        out_shape=jax.ShapeDtypeStruct((E_pad, 128), jnp.float32),
    )(rplane, splane, eW0eff, eb0.reshape(1, 128), eW1, eb1.reshape(1, 128),
      eW2, eb2.reshape(1, 128), eW3, eb3.reshape(1, 128))

    recv_perm = recv_p.reshape(E_pad // BE, BE // 8, 8)
    recv_perm = recv_perm.transpose(0, 2, 1).reshape(-1)
    zeros_rows = jnp.zeros((ROWS_PER_SUB, 128), jnp.float32)
    scatter = functools.partial(
        pl.kernel,
        out_type=jax.ShapeDtypeStruct((NC, NP, 128), jnp.float32),
        mesh=mesh,
        scratch_types=[
            pltpu.VMEM_SHARED((NP, 128), jnp.float32),
            pltpu.VMEM((CH, 128), jnp.float32),
            pltpu.VMEM((CH,), jnp.int32),
        ],
    )(functools.partial(_scatter_kernel, K))
    partials = scatter(marsh, recv_perm, zeros_rows)

    BN = 2000
    p0 = partials[0, :N]
    p1 = partials[1, :N]
    nW0a = jnp.pad(nW0[:7], ((0, 1), (0, 0)))
    nW0b = nW0[7:]
    nW3p = jnp.pad(nW3, ((0, 0), (0, 5)))
    nb3p = jnp.pad(nb3, ((0, 5),)).reshape(1, 8)
    out8 = pl.pallas_call(
        _node_mlp_kernel,
        grid=(N // BN,),
        in_specs=[
            pl.BlockSpec((BN, 6), lambda i: (i, 0)),
            pl.BlockSpec((BN, 1), lambda i: (i, 0)),
            pl.BlockSpec((BN, 128), lambda i: (i, 0)),
            pl.BlockSpec((BN, 128), lambda i: (i, 0)),
            pl.BlockSpec((8, 128), lambda i: (0, 0)),
            pl.BlockSpec((128, 128), lambda i: (0, 0)),
            pl.BlockSpec((1, 128), lambda i: (0, 0)),
            pl.BlockSpec((128, 128), lambda i: (0, 0)),
            pl.BlockSpec((1, 128), lambda i: (0, 0)),
            pl.BlockSpec((128, 128), lambda i: (0, 0)),
            pl.BlockSpec((1, 128), lambda i: (0, 0)),
            pl.BlockSpec((128, 8), lambda i: (0, 0)),
            pl.BlockSpec((1, 8), lambda i: (0, 0)),
        ],
        out_specs=pl.BlockSpec((BN, 8), lambda i: (i, 0)),
        out_shape=jax.ShapeDtypeStruct((N, 8), jnp.float32),
    )(x, ids, p0, p1, nW0a, nW0b, nb0.reshape(1, 128),
      nW1, nb1.reshape(1, 128), nW2, nb2.reshape(1, 128), nW3p, nb3p)
    return out8[:, :6]


# speedup vs baseline: 5.1963x; 1.2939x over previous
import functools

import jax
import jax.numpy as jnp
from jax import lax
from jax.experimental import pallas as pl
from jax.experimental.pallas import tpu as pltpu
from jax.experimental.pallas import tpu_sc as plsc

N = 10000
DZ = -0.0106666666666666
DT = 0.01

NC = 2
NS = 16
NW = NC * NS
CH = 128
NP = N + 112
ROWS_PER_SUB = NP // NS


def _prep_kernel(x_ref, ids_ref, o_ref):
    x = x_ref[...]
    ids = ids_ref[...]
    xb = jnp.concatenate(
        [x[:, 0:2], x[:, 2:3] - ids * jnp.float32(DZ), x[:, 3:6], ids,
         jnp.zeros((x.shape[0], 9), jnp.float32)], axis=1)
    o_ref[...] = jnp.concatenate(
        [xb, jnp.zeros((16, 16), jnp.float32)], axis=0)


def _gather_kernel(K, xp_hbm, send_hbm, recv_hbm, rout_hbm, sout_hbm,
                   sidxs, ridxs, sbufs, rbufs, gsems, wsems):
    wid = lax.axis_index("s") * NC + lax.axis_index("c")
    pltpu.sync_copy(send_hbm.at[pl.ds(wid * K, K)], sidxs)
    pltpu.sync_copy(recv_hbm.at[pl.ds(wid * K, K)], ridxs)

    def gath(k, b):
        pltpu.async_copy(xp_hbm.at[sidxs.at[k]], sbufs[b], gsems[2 * b])
        pltpu.async_copy(xp_hbm.at[ridxs.at[k]], rbufs[b], gsems[2 * b + 1])

    def wait_g(b):
        pltpu.make_async_copy(xp_hbm.at[sidxs.at[0]], sbufs[b],
                              gsems[2 * b]).wait()
        pltpu.make_async_copy(xp_hbm.at[ridxs.at[0]], rbufs[b],
                              gsems[2 * b + 1]).wait()

    def write(k, b):
        off = (wid * K + k) * CH
        pltpu.async_copy(rbufs[b], rout_hbm.at[pl.ds(off, CH)], wsems[2 * b])
        pltpu.async_copy(sbufs[b], sout_hbm.at[pl.ds(off, CH)],
                         wsems[2 * b + 1])

    def wait_w(b):
        pltpu.make_async_copy(rbufs[b], rout_hbm.at[pl.ds(0, CH)],
                              wsems[2 * b]).wait()
        pltpu.make_async_copy(sbufs[b], sout_hbm.at[pl.ds(0, CH)],
                              wsems[2 * b + 1]).wait()

    gath(0, 0)
    gath(1, 1)

    def body(j, _):
        k0 = j * 4
        for b in range(4):
            k = k0 + b
            wait_g(b)
            write(k, b)
            nb = (b + 2) % 4
            nk = k + 2

            @pl.when(nk < K)
            def _():
                @pl.when(nk >= 4)
                def _():
                    wait_w(nb)
                gath(nk, nb)
        return 0

    lax.fori_loop(0, K // 4, body, 0)
    for b in range(4):
        wait_w(b)


def _edge_mlp_kernel(r_ref, s_ref, w0, b0, w1, b1, w2, b2, w3, b3, o_ref):
    d = r_ref[...] - s_ref[...]
    h = jnp.concatenate(
        [jnp.dot(d[:, 16 * q:16 * q + 16], w0[...],
                 preferred_element_type=jnp.float32) for q in range(8)],
        axis=0)
    h = jnp.maximum(h + b0[...], 0.0)
    h = jnp.dot(h, w1[...], preferred_element_type=jnp.float32)
    h = jnp.maximum(h + b1[...], 0.0)
    h = jnp.dot(h, w2[...], preferred_element_type=jnp.float32)
    h = jnp.maximum(h + b2[...], 0.0)
    h = jnp.dot(h, w3[...], preferred_element_type=jnp.float32)
    o_ref[...] = h + b3[...]


def _scatter_kernel(K, marsh_hbm, recv_hbm, zeros_hbm, out_hbm,
                    shared, idxs, mbufs, lsems, asems):
    c = lax.axis_index("c")
    s = lax.axis_index("s")
    wid = s * NC + c
    r0 = s * ROWS_PER_SUB
    pltpu.sync_copy(zeros_hbm, shared.at[pl.ds(r0, ROWS_PER_SUB)])
    pltpu.sync_copy(recv_hbm.at[pl.ds(wid * K, K)], idxs)
    plsc.subcore_barrier()

    def load(k, b):
        pltpu.async_copy(marsh_hbm.at[pl.ds((wid * K + k) * CH, CH)],
                         mbufs[b], lsems[b])

    def wait_l(b):
        pltpu.make_async_copy(marsh_hbm.at[pl.ds(0, CH)], mbufs[b],
                              lsems[b]).wait()

    def add(k, b):
        pltpu.async_copy(mbufs[b], shared.at[idxs.at[k]], asems[b], add=True)

    def wait_a(b):
        pltpu.make_async_copy(mbufs[b], shared.at[idxs.at[0]],
                              asems[b]).wait()

    load(0, 0)
    load(1, 1)

    def body(j, _):
        k0 = j * 2
        for b in range(2):
            k = k0 + b
            wait_l(b)
            add(k, b)

            @pl.when(k + 2 < K)
            def _():
                wait_a(b)
                load(k + 2, b)
        return 0

    lax.fori_loop(0, K // 2, body, 0)
    wait_a(0)
    wait_a(1)
    plsc.subcore_barrier()
    pltpu.sync_copy(shared.at[pl.ds(r0, ROWS_PER_SUB)],
                    out_hbm.at[c, pl.ds(r0, ROWS_PER_SUB)])


def _node_mlp_kernel(x_ref, ids_ref, p0_ref, p1_ref,
                     w0a, w0b, b0, w1, b1, w2, b2, w3, b3, o_ref):
    x = x_ref[...]
    ids = ids_ref[...]
    rest = jnp.concatenate(
        [jnp.zeros_like(ids), jnp.zeros_like(ids), ids * jnp.float32(DZ),
         jnp.zeros_like(ids), jnp.zeros_like(ids), jnp.zeros_like(ids)],
        axis=1)
    xb = jnp.concatenate([x - rest, ids, jnp.zeros_like(ids)], axis=1)
    aggr = p0_ref[...] + p1_ref[...]
    h = (jnp.dot(xb, w0a[...], preferred_element_type=jnp.float32)
         + jnp.dot(aggr, w0b[...], preferred_element_type=jnp.float32))
    h = jnp.maximum(h + b0[...], 0.0)
    h = jnp.dot(h, w1[...], preferred_element_type=jnp.float32)
    h = jnp.maximum(h + b1[...], 0.0)
    h = jnp.dot(h, w2[...], preferred_element_type=jnp.float32)
    h = jnp.maximum(h + b2[...], 0.0)
    d8 = jnp.dot(h, w3[...], preferred_element_type=jnp.float32) + b3[...]
    v = x[:, 3:6] + d8[:, 0:3]
    xn = x[:, 0:3] + v * jnp.float32(DT)
    o_ref[...] = jnp.concatenate([xn, v, jnp.zeros_like(v[:, 0:2])], axis=1)


def kernel(x, ids, edge_index, eW0, eb0, eW1, eb1, eW2, eb2, eW3, eb3,
           nW0, nb0, nW1, nb1, nW2, nb2, nW3, nb3):
    E = edge_index.shape[1]
    K = -(-E // (NW * CH))
    K = -(-K // 4) * 4
    E_pad = NW * CH * K

    send = edge_index[0]
    recv = edge_index[1]
    pad = jnp.full((E_pad - E,), N, jnp.int32)
    send_p = jnp.concatenate([send, pad]).reshape(NW * K, CH)
    recv_p = jnp.concatenate([recv, pad])

    xp = pl.pallas_call(
        _prep_kernel,
        out_shape=jax.ShapeDtypeStruct((N + 16, 16), jnp.float32),
    )(x, ids)

    mesh = plsc.VectorSubcoreMesh(core_axis_name="c", subcore_axis_name="s",
                                  num_cores=NC, num_subcores=NS)
    gather = functools.partial(
        pl.kernel,
        out_type=[jax.ShapeDtypeStruct((E_pad, 16), jnp.float32),
                  jax.ShapeDtypeStruct((E_pad, 16), jnp.float32)],
        mesh=mesh,
        compiler_params=pltpu.CompilerParams(use_tc_tiling_on_sc=False),
        scratch_types=[
            pltpu.VMEM((K, CH), jnp.int32),
            pltpu.VMEM((K, CH), jnp.int32),
            [pltpu.VMEM((CH, 16), jnp.float32) for _ in range(4)],
            [pltpu.VMEM((CH, 16), jnp.float32) for _ in range(4)],
            [pltpu.SemaphoreType.DMA for _ in range(8)],
            [pltpu.SemaphoreType.DMA for _ in range(8)],
        ],
    )(functools.partial(_gather_kernel, K))
    rout, sout = gather(xp, send_p, recv_p.reshape(NW * K, CH))
    rplane = rout.reshape(E_pad // 8, 128)
    splane = sout.reshape(E_pad // 8, 128)

    eW0eff = jnp.pad(eW0, ((0, 9), (0, 0)))

    BE = 2048
    grid = E_pad // BE
    marsh = pl.pallas_call(
        _edge_mlp_kernel,
        grid=(grid,),
        in_specs=[
            pl.BlockSpec((BE // 8, 128), lambda i: (i, 0)),
            pl.BlockSpec((BE // 8, 128), lambda i: (i, 0)),
            pl.BlockSpec((16, 128), lambda i: (0, 0)),
            pl.BlockSpec((1, 128), lambda i: (0, 0)),
            pl.BlockSpec((128, 128), lambda i: (0, 0)),
            pl.BlockSpec((1, 128), lambda i: (0, 0)),
            pl.BlockSpec((128, 128), lambda i: (0, 0)),
            pl.BlockSpec((1, 128), lambda i: (0, 0)),
            pl.BlockSpec((128, 128), lambda i: (0, 0)),
            pl.BlockSpec((1, 128), lambda i: (0, 0)),
        ],
        out_specs=pl.BlockSpec((BE, 128), lambda i: (i, 0)),
        out_shape=jax.ShapeDtypeStruct((E_pad, 128), jnp.float32),
    )(rplane, splane, eW0eff, eb0.reshape(1, 128), eW1, eb1.reshape(1, 128),
      eW2, eb2.reshape(1, 128), eW3, eb3.reshape(1, 128))

    recv_perm = recv_p.reshape(E_pad // BE, BE // 8, 8)
    recv_perm = recv_perm.transpose(0, 2, 1).reshape(NW * K, CH)
    zeros_rows = jnp.zeros((ROWS_PER_SUB, 128), jnp.float32)
    scatter = functools.partial(
        pl.kernel,
        out_type=jax.ShapeDtypeStruct((NC, NP, 128), jnp.float32),
        mesh=mesh,
        scratch_types=[
            pltpu.VMEM_SHARED((NP, 128), jnp.float32),
            pltpu.VMEM((K, CH), jnp.int32),
            [pltpu.VMEM((CH, 128), jnp.float32) for _ in range(2)],
            [pltpu.SemaphoreType.DMA for _ in range(2)],
            [pltpu.SemaphoreType.DMA for _ in range(2)],
        ],
    )(functools.partial(_scatter_kernel, K))
    partials = scatter(marsh, recv_perm, zeros_rows)

    BN = 2000
    p0 = partials[0, :N]
    p1 = partials[1, :N]
    nW0a = jnp.pad(nW0[:7], ((0, 1), (0, 0)))
    nW0b = nW0[7:]
    nW3p = jnp.pad(nW3, ((0, 0), (0, 5)))
    nb3p = jnp.pad(nb3, ((0, 5),)).reshape(1, 8)
    out8 = pl.pallas_call(
        _node_mlp_kernel,
        grid=(N // BN,),
        in_specs=[
            pl.BlockSpec((BN, 6), lambda i: (i, 0)),
            pl.BlockSpec((BN, 1), lambda i: (i, 0)),
            pl.BlockSpec((BN, 128), lambda i: (i, 0)),
            pl.BlockSpec((BN, 128), lambda i: (i, 0)),
            pl.BlockSpec((8, 128), lambda i: (0, 0)),
            pl.BlockSpec((128, 128), lambda i: (0, 0)),
            pl.BlockSpec((1, 128), lambda i: (0, 0)),
            pl.BlockSpec((128, 128), lambda i: (0, 0)),
            pl.BlockSpec((1, 128), lambda i: (0, 0)),
            pl.BlockSpec((128, 128), lambda i: (0, 0)),
            pl.BlockSpec((1, 128), lambda i: (0, 0)),
            pl.BlockSpec((128, 8), lambda i: (0, 0)),
            pl.BlockSpec((1, 8), lambda i: (0, 0)),
        ],
        out_specs=pl.BlockSpec((BN, 8), lambda i: (i, 0)),
        out_shape=jax.ShapeDtypeStruct((N, 8), jnp.float32),
    )(x, ids, p0, p1, nW0a, nW0b, nb0.reshape(1, 128),
      nW1, nb1.reshape(1, 128), nW2, nb2.reshape(1, 128), nW3p, nb3p)
    return out8[:, :6]


# speedup vs baseline: 5.6860x; 1.0942x over previous
import functools

import jax
import jax.numpy as jnp
from jax import lax
from jax.experimental import pallas as pl
from jax.experimental.pallas import tpu as pltpu
from jax.experimental.pallas import tpu_sc as plsc

N = 10000
DZ = -0.0106666666666666
DT = 0.01

NC = 2
NS = 16
NW = NC * NS
CH = 128
NP = N + 112
ROWS_PER_SUB = NP // NS


def _prep_kernel(x_ref, ids_ref, o_ref):
    x = x_ref[...]
    ids = ids_ref[...]
    xb = jnp.concatenate(
        [x[:, 0:2], x[:, 2:3] - ids * jnp.float32(DZ), x[:, 3:6], ids,
         jnp.zeros((x.shape[0], 9), jnp.float32)], axis=1)
    o_ref[...] = jnp.concatenate(
        [xb, jnp.zeros((16, 16), jnp.float32)], axis=0)


def _gather_kernel(K, xp_hbm, send_hbm, recv_hbm, rout_hbm, sout_hbm,
                   sidxs, ridxs, sbufs, rbufs, gsems, wsems):
    wid = lax.axis_index("s") * NC + lax.axis_index("c")
    pltpu.sync_copy(send_hbm.at[pl.ds(wid * K, K)], sidxs)
    pltpu.sync_copy(recv_hbm.at[pl.ds(wid * K, K)], ridxs)

    def gath(k, b):
        pltpu.async_copy(xp_hbm.at[sidxs.at[k]], sbufs[b], gsems[2 * b])
        pltpu.async_copy(xp_hbm.at[ridxs.at[k]], rbufs[b], gsems[2 * b + 1])

    def wait_g(b):
        pltpu.make_async_copy(xp_hbm.at[sidxs.at[0]], sbufs[b],
                              gsems[2 * b]).wait()
        pltpu.make_async_copy(xp_hbm.at[ridxs.at[0]], rbufs[b],
                              gsems[2 * b + 1]).wait()

    def write(k, b):
        off = (wid * K + k) * CH
        pltpu.async_copy(rbufs[b], rout_hbm.at[pl.ds(off, CH)], wsems[2 * b])
        pltpu.async_copy(sbufs[b], sout_hbm.at[pl.ds(off, CH)],
                         wsems[2 * b + 1])

    def wait_w(b):
        pltpu.make_async_copy(rbufs[b], rout_hbm.at[pl.ds(0, CH)],
                              wsems[2 * b]).wait()
        pltpu.make_async_copy(sbufs[b], sout_hbm.at[pl.ds(0, CH)],
                              wsems[2 * b + 1]).wait()

    gath(0, 0)
    gath(1, 1)

    def body(j, _):
        k0 = j * 4
        for b in range(4):
            k = k0 + b
            wait_g(b)
            write(k, b)
            nb = (b + 2) % 4
            nk = k + 2

            @pl.when(nk < K)
            def _():
                @pl.when(nk >= 4)
                def _():
                    wait_w(nb)
                gath(nk, nb)
        return 0

    lax.fori_loop(0, K // 4, body, 0)
    for b in range(4):
        wait_w(b)


def _edge_mlp_kernel(r_ref, s_ref, w0, b0, w1, b1, w2, b2, w3, b3, o_ref):
    d = r_ref[...] - s_ref[...]
    h = jnp.concatenate(
        [jnp.dot(d[:, 16 * q:16 * q + 16], w0[...],
                 preferred_element_type=jnp.float32) for q in range(8)],
        axis=0)
    h = jnp.maximum(h + b0[...], 0.0)
    h = jnp.dot(h, w1[...], preferred_element_type=jnp.float32)
    h = jnp.maximum(h + b1[...], 0.0)
    h = jnp.dot(h, w2[...], preferred_element_type=jnp.float32)
    h = jnp.maximum(h + b2[...], 0.0)
    h = jnp.dot(h, w3[...], preferred_element_type=jnp.float32)
    o_ref[...] = h + b3[...]


def _scatter_kernel(K, marsh_hbm, recv_hbm, zeros_hbm, out_hbm,
                    shared, idxs, mbufs, lsems, asems):
    c = lax.axis_index("c")
    s = lax.axis_index("s")
    wid = s * NC + c
    r0 = s * ROWS_PER_SUB
    pltpu.sync_copy(zeros_hbm, shared.at[pl.ds(r0, ROWS_PER_SUB)])
    pltpu.sync_copy(recv_hbm.at[pl.ds(wid * K, K)], idxs)
    plsc.subcore_barrier()

    def load(k, b):
        pltpu.async_copy(marsh_hbm.at[pl.ds((wid * K + k) * CH, CH)],
                         mbufs[b], lsems[b])

    def wait_l(b):
        pltpu.make_async_copy(marsh_hbm.at[pl.ds(0, CH)], mbufs[b],
                              lsems[b]).wait()

    def add(k, b):
        pltpu.async_copy(mbufs[b], shared.at[idxs.at[k]], asems[b], add=True)

    def wait_a(b):
        pltpu.make_async_copy(mbufs[b], shared.at[idxs.at[0]],
                              asems[b]).wait()

    load(0, 0)
    load(1, 1)

    def body(j, _):
        k0 = j * 2
        for b in range(2):
            k = k0 + b
            wait_l(b)
            add(k, b)

            @pl.when(k + 2 < K)
            def _():
                wait_a(b)
                load(k + 2, b)
        return 0

    lax.fori_loop(0, K // 2, body, 0)
    wait_a(0)
    wait_a(1)
    plsc.subcore_barrier()
    pltpu.sync_copy(shared.at[pl.ds(r0, ROWS_PER_SUB)],
                    out_hbm.at[c, pl.ds(r0, ROWS_PER_SUB)])


def _node_mlp_kernel(x_ref, ids_ref, p0_ref, p1_ref,
                     w0a, w0b, b0, w1, b1, w2, b2, w3, b3, o_ref):
    x = x_ref[...]
    ids = ids_ref[...]
    rest = jnp.concatenate(
        [jnp.zeros_like(ids), jnp.zeros_like(ids), ids * jnp.float32(DZ),
         jnp.zeros_like(ids), jnp.zeros_like(ids), jnp.zeros_like(ids)],
        axis=1)
    xb = jnp.concatenate([x - rest, ids, jnp.zeros_like(ids)], axis=1)
    aggr = p0_ref[...] + p1_ref[...]
    h = (jnp.dot(xb, w0a[...], preferred_element_type=jnp.float32)
         + jnp.dot(aggr, w0b[...], preferred_element_type=jnp.float32))
    h = jnp.maximum(h + b0[...], 0.0)
    h = jnp.dot(h, w1[...], preferred_element_type=jnp.float32)
    h = jnp.maximum(h + b1[...], 0.0)
    h = jnp.dot(h, w2[...], preferred_element_type=jnp.float32)
    h = jnp.maximum(h + b2[...], 0.0)
    d8 = jnp.dot(h, w3[...], preferred_element_type=jnp.float32) + b3[...]
    v = x[:, 3:6] + d8[:, 0:3]
    xn = x[:, 0:3] + v * jnp.float32(DT)
    o_ref[...] = jnp.concatenate([xn, v, jnp.zeros_like(v[:, 0:2])], axis=1)


def kernel(x, ids, edge_index, eW0, eb0, eW1, eb1, eW2, eb2, eW3, eb3,
           nW0, nb0, nW1, nb1, nW2, nb2, nW3, nb3):
    E = edge_index.shape[1]
    K = -(-E // (NW * CH))
    K = -(-K // 8) * 8
    E_pad = NW * CH * K

    send = edge_index[0]
    recv = edge_index[1]
    pad = jnp.full((E_pad - E,), N, jnp.int32)
    send_p = jnp.concatenate([send, pad]).reshape(NW * K, CH)
    recv_p = jnp.concatenate([recv, pad])

    xp = pl.pallas_call(
        _prep_kernel,
        out_shape=jax.ShapeDtypeStruct((N + 16, 16), jnp.float32),
    )(x, ids)

    mesh = plsc.VectorSubcoreMesh(core_axis_name="c", subcore_axis_name="s",
                                  num_cores=NC, num_subcores=NS)
    gather = functools.partial(
        pl.kernel,
        out_type=[jax.ShapeDtypeStruct((E_pad, 16), jnp.float32),
                  jax.ShapeDtypeStruct((E_pad, 16), jnp.float32)],
        mesh=mesh,
        compiler_params=pltpu.CompilerParams(use_tc_tiling_on_sc=False),
        scratch_types=[
            pltpu.VMEM((K, CH), jnp.int32),
            pltpu.VMEM((K, CH), jnp.int32),
            [pltpu.VMEM((CH, 16), jnp.float32) for _ in range(4)],
            [pltpu.VMEM((CH, 16), jnp.float32) for _ in range(4)],
            [pltpu.SemaphoreType.DMA for _ in range(8)],
            [pltpu.SemaphoreType.DMA for _ in range(8)],
        ],
    )(functools.partial(_gather_kernel, K))
    rout, sout = gather(xp, send_p, recv_p.reshape(NW * K, CH))
    rplane = rout.reshape(E_pad // 8, 128)
    splane = sout.reshape(E_pad // 8, 128)

    eW0eff = jnp.pad(eW0, ((0, 9), (0, 0)))

    BE = 4096
    grid = E_pad // BE
    marsh = pl.pallas_call(
        _edge_mlp_kernel,
        grid=(grid,),
        in_specs=[
            pl.BlockSpec((BE // 8, 128), lambda i: (i, 0)),
            pl.BlockSpec((BE // 8, 128), lambda i: (i, 0)),
            pl.BlockSpec((16, 128), lambda i: (0, 0)),
            pl.BlockSpec((1, 128), lambda i: (0, 0)),
            pl.BlockSpec((128, 128), lambda i: (0, 0)),
            pl.BlockSpec((1, 128), lambda i: (0, 0)),
            pl.BlockSpec((128, 128), lambda i: (0, 0)),
            pl.BlockSpec((1, 128), lambda i: (0, 0)),
            pl.BlockSpec((128, 128), lambda i: (0, 0)),
            pl.BlockSpec((1, 128), lambda i: (0, 0)),
        ],
        out_specs=pl.BlockSpec((BE, 128), lambda i: (i, 0)),
        out_shape=jax.ShapeDtypeStruct((E_pad, 128), jnp.float32),
    )(rplane, splane, eW0eff, eb0.reshape(1, 128), eW1, eb1.reshape(1, 128),
      eW2, eb2.reshape(1, 128), eW3, eb3.reshape(1, 128))

    recv_perm = recv_p.reshape(E_pad // BE, BE // 8, 8)
    recv_perm = recv_perm.transpose(0, 2, 1).reshape(NW * K, CH)
    zeros_rows = jnp.zeros((ROWS_PER_SUB, 128), jnp.float32)
    scatter = functools.partial(
        pl.kernel,
        out_type=jax.ShapeDtypeStruct((NC, NP, 128), jnp.float32),
        mesh=mesh,
        scratch_types=[
            pltpu.VMEM_SHARED((NP, 128), jnp.float32),
            pltpu.VMEM((K, CH), jnp.int32),
            [pltpu.VMEM((CH, 128), jnp.float32) for _ in range(2)],
            [pltpu.SemaphoreType.DMA for _ in range(2)],
            [pltpu.SemaphoreType.DMA for _ in range(2)],
        ],
    )(functools.partial(_scatter_kernel, K))
    partials = scatter(marsh, recv_perm, zeros_rows)

    BN = 2000
    p0 = partials[0, :N]
    p1 = partials[1, :N]
    nW0a = jnp.pad(nW0[:7], ((0, 1), (0, 0)))
    nW0b = nW0[7:]
    nW3p = jnp.pad(nW3, ((0, 0), (0, 5)))
    nb3p = jnp.pad(nb3, ((0, 5),)).reshape(1, 8)
    out8 = pl.pallas_call(
        _node_mlp_kernel,
        grid=(N // BN,),
        in_specs=[
            pl.BlockSpec((BN, 6), lambda i: (i, 0)),
            pl.BlockSpec((BN, 1), lambda i: (i, 0)),
            pl.BlockSpec((BN, 128), lambda i: (i, 0)),
            pl.BlockSpec((BN, 128), lambda i: (i, 0)),
            pl.BlockSpec((8, 128), lambda i: (0, 0)),
            pl.BlockSpec((128, 128), lambda i: (0, 0)),
            pl.BlockSpec((1, 128), lambda i: (0, 0)),
            pl.BlockSpec((128, 128), lambda i: (0, 0)),
            pl.BlockSpec((1, 128), lambda i: (0, 0)),
            pl.BlockSpec((128, 128), lambda i: (0, 0)),
            pl.BlockSpec((1, 128), lambda i: (0, 0)),
            pl.BlockSpec((128, 8), lambda i: (0, 0)),
            pl.BlockSpec((1, 8), lambda i: (0, 0)),
        ],
        out_specs=pl.BlockSpec((BN, 8), lambda i: (i, 0)),
        out_shape=jax.ShapeDtypeStruct((N, 8), jnp.float32),
    )(x, ids, p0, p1, nW0a, nW0b, nb0.reshape(1, 128),
      nW1, nb1.reshape(1, 128), nW2, nb2.reshape(1, 128), nW3p, nb3p)
    return out8[:, :6]
